# baseline (speedup 1.0000x reference)
import functools

import jax
import jax.numpy as jnp
from jax import lax
from jax.experimental import pallas as pl
from jax.experimental.pallas import tpu as pltpu
from jax.experimental.pallas import tpu_sc as plsc

_VOCAB = 1000
_VSTRIDE = 1000
_BATCH = 16384
_NC = 2
_NS = 16
_NW = _NC * _NS
_VPW = 32
_NOCT = 4
_BC = 512
_NBC = _BATCH // _BC

_mesh = plsc.VectorSubcoreMesh(core_axis_name="c", subcore_axis_name="s")


@functools.partial(
    pl.kernel,
    out_type=jax.ShapeDtypeStruct((_VOCAB, _BATCH), jnp.float32),
    mesh=_mesh,
    scratch_types=[
        pltpu.VMEM((_BATCH,), jnp.int32),
        pltpu.VMEM((_VPW * _VSTRIDE,), jnp.float32),
        pltpu.VMEM((2, _VPW, _BC), jnp.float32),
        pltpu.SemaphoreType.DMA,
        pltpu.SemaphoreType.DMA,
    ],
    compiler_params=pltpu.CompilerParams(needs_layout_passes=False),
)
def _tgather_kernel(tab_hbm, idx_hbm, out_hbm, idx_l, tab_l, bufs, sem0, sem1):
    wid = lax.axis_index("s") * _NC + lax.axis_index("c")
    v0 = wid * _VPW
    sems = (sem0, sem1)
    pltpu.sync_copy(idx_hbm, idx_l)
    for k in range(_NOCT):
        @pl.when(v0 + k * 8 < _VOCAB)
        def _load(k=k):
            pltpu.sync_copy(
                tab_hbm.at[pl.ds((v0 + k * 8) * _VSTRIDE, 8 * _VSTRIDE)],
                tab_l.at[pl.ds(k * 8 * _VSTRIDE, 8 * _VSTRIDE)],
            )

    def store_descs(par, b0):
        descs = []
        for k in range(_NOCT):
            descs.append((
                v0 + k * 8 < _VOCAB,
                bufs.at[par, pl.ds(k * 8, 8)],
                out_hbm.at[pl.ds(v0 + k * 8, 8), pl.ds(b0, _BC)],
                sems[par],
            ))
        return descs

    def do_chunk(bc2, par, drain):
        b0 = pl.multiple_of((2 * bc2 + par) * _BC, _BC)

        @pl.when(drain)
        def _drain():
            for cond, src, dst, sem in store_descs(par, b0):
                @pl.when(cond)
                def _w(src=src, dst=dst, sem=sem):
                    pltpu.make_async_copy(src, dst, sem).wait()

        @plsc.parallel_loop(0, _BC // 16, unroll=1)
        def body(b16):
            idxv = idx_l[pl.ds(b0 + b16 * 16, 16)]
            for vl in range(_VPW):
                x = plsc.load_gather(tab_l, [idxv + vl * _VSTRIDE])
                bufs[par, vl, pl.ds(b16 * 16, 16)] = x

        for cond, src, dst, sem in store_descs(par, b0):
            @pl.when(cond)
            def _s(src=src, dst=dst, sem=sem):
                pltpu.async_copy(src, dst, sem)

    def chunk_body(bc2, carry):
        do_chunk(bc2, 0, bc2 >= 1)
        do_chunk(bc2, 1, bc2 >= 1)
        return carry

    lax.fori_loop(0, _NBC // 2, chunk_body, 0)

    for par in (0, 1):
        b0 = (_NBC - 2 + par) * _BC
        for cond, src, dst, sem in store_descs(par, b0):
            @pl.when(cond)
            def _w(src=src, dst=dst, sem=sem):
                pltpu.make_async_copy(src, dst, sem).wait()


def kernel(idx, embedding):
    return _tgather_kernel(embedding.T.reshape(-1), idx).T


# speedup vs baseline: 1.0248x; 1.0248x over previous
import functools

import jax
import jax.numpy as jnp
from jax import lax
from jax.experimental import pallas as pl
from jax.experimental.pallas import tpu as pltpu
from jax.experimental.pallas import tpu_sc as plsc

_VOCAB = 1000
_VSTRIDE = 1000
_NTROW = _VOCAB // 8
_BATCH = 16384
_NC = 2
_NS = 16
_NW = _NC * _NS
_VPW = 32
_NOCT = 4
_BC = 1024
_NBC = _BATCH // _BC

_mesh = plsc.VectorSubcoreMesh(core_axis_name="c", subcore_axis_name="s")


@functools.partial(
    pl.kernel,
    out_type=jax.ShapeDtypeStruct((_NTROW, 8, _BATCH), jnp.float32),
    mesh=_mesh,
    scratch_types=[
        pltpu.VMEM((_BATCH,), jnp.int32),
        pltpu.VMEM((_VPW * _VSTRIDE,), jnp.float32),
        pltpu.VMEM((2, _NOCT, 8, _BC), jnp.float32),
        pltpu.SemaphoreType.DMA,
        pltpu.SemaphoreType.DMA,
    ],
    compiler_params=pltpu.CompilerParams(needs_layout_passes=False),
)
def _tgather_kernel(tab_hbm, idx_hbm, out_hbm, idx_l, tab_l, bufs, sem0, sem1):
    wid = lax.axis_index("s") * _NC + lax.axis_index("c")
    v0 = wid * _VPW
    t0 = wid * _NOCT
    full = t0 + _NOCT <= _NTROW
    sems = (sem0, sem1)
    pltpu.sync_copy(idx_hbm, idx_l)
    for k in range(_NOCT):
        @pl.when(t0 + k < _NTROW)
        def _load(k=k):
            pltpu.sync_copy(
                tab_hbm.at[pl.ds((v0 + k * 8) * _VSTRIDE, 8 * _VSTRIDE)],
                tab_l.at[pl.ds(k * 8 * _VSTRIDE, 8 * _VSTRIDE)],
            )

    def store_descs(par, b0):
        descs = [(
            full,
            bufs.at[par],
            out_hbm.at[pl.ds(t0, _NOCT), :, pl.ds(b0, _BC)],
            sems[par],
        )]
        for k in range(_NOCT):
            descs.append((
                jnp.logical_and(jnp.logical_not(full), t0 + k < _NTROW),
                bufs.at[par, k],
                out_hbm.at[t0 + k, :, pl.ds(b0, _BC)],
                sems[par],
            ))
        return descs

    def do_chunk(bc2, par, drain):
        b0 = pl.multiple_of((2 * bc2 + par) * _BC, _BC)

        @pl.when(drain)
        def _drain():
            for cond, src, dst, sem in store_descs(par, b0):
                @pl.when(cond)
                def _w(src=src, dst=dst, sem=sem):
                    pltpu.make_async_copy(src, dst, sem).wait()

        @plsc.parallel_loop(0, _BC // 16, unroll=1)
        def body(b16):
            idxv = idx_l[pl.ds(b0 + b16 * 16, 16)]
            for vl in range(_VPW):
                x = plsc.load_gather(tab_l, [idxv + vl * _VSTRIDE])
                bufs[par, vl // 8, vl % 8, pl.ds(b16 * 16, 16)] = x

        for cond, src, dst, sem in store_descs(par, b0):
            @pl.when(cond)
            def _s(src=src, dst=dst, sem=sem):
                pltpu.async_copy(src, dst, sem)

    def chunk_body(bc2, carry):
        do_chunk(bc2, 0, bc2 >= 1)
        do_chunk(bc2, 1, bc2 >= 1)
        return carry

    lax.fori_loop(0, _NBC // 2, chunk_body, 0)

    for par in (0, 1):
        b0 = (_NBC - 2 + par) * _BC
        for cond, src, dst, sem in store_descs(par, b0):
            @pl.when(cond)
            def _w(src=src, dst=dst, sem=sem):
                pltpu.make_async_copy(src, dst, sem).wait()


def kernel(idx, embedding):
    out3 = _tgather_kernel(embedding.T.reshape(-1), idx)
    return out3.reshape(_VOCAB, _BATCH).T
